# baseline (speedup 1.0000x reference)
import functools

import jax
import jax.numpy as jnp
from jax import lax
from jax.experimental import pallas as pl
from jax.experimental.pallas import tpu as pltpu
from jax.experimental.pallas import tpu_sc as plsc

N = 10000
E = 320000
IN = 128
HID = 256
OUT = 384
HEADS = 3
NPAD = 12288
NC, NS, L = 2, 16, 16
CH = 2000
FB = 64
F32 = jnp.float32
I32 = jnp.int32


def _mm_alpha(x, W, Amat, bm=1024):
    m, k = x.shape
    c = W.shape[1]

    def body(x_ref, w_ref, a_ref, y_ref, al_ref):
        h = jnp.dot(x_ref[...], w_ref[...], preferred_element_type=F32)
        y_ref[...] = h
        al_ref[...] = jnp.dot(h, a_ref[...], preferred_element_type=F32)

    return pl.pallas_call(
        body,
        grid=(m // bm,),
        in_specs=[
            pl.BlockSpec((bm, k), lambda i: (i, 0)),
            pl.BlockSpec((k, c), lambda i: (0, 0)),
            pl.BlockSpec((c, 16), lambda i: (0, 0)),
        ],
        out_specs=[
            pl.BlockSpec((bm, c), lambda i: (i, 0)),
            pl.BlockSpec((bm, 16), lambda i: (i, 0)),
        ],
        out_shape=[
            jax.ShapeDtypeStruct((m, c), F32),
            jax.ShapeDtypeStruct((m, 16), F32),
        ],
    )(x, W, Amat)


def _rinv_self(s32, al_a, al_b):
    H = al_a.shape[0]

    def body(s_ref, aa_ref, ab_ref, rv_ref, as_ref):
        es = aa_ref[...] + ab_ref[...]
        ps = jnp.exp(jnp.where(es >= 0.0, es, es * 0.2))
        st = jnp.sum(s_ref[...], axis=0) + ps
        rv = 1.0 / (st + 1e-16)
        rv_ref[...] = rv
        as_ref[...] = ps * rv

    return pl.pallas_call(
        body,
        out_shape=[
            jax.ShapeDtypeStruct((H, NPAD), F32),
            jax.ShapeDtypeStruct((H, NPAD), F32),
        ],
    )(s32, al_a, al_b)


def _combine(aggP, h, asel16, bvec, SEL, bm=128):
    m, crow = h.shape
    npart = aggP.shape[0]

    def body(g_ref, h_ref, a_ref, s_ref, b_ref, o_ref):
        af = jnp.dot(a_ref[...], s_ref[...], preferred_element_type=F32)
        g = jnp.sum(g_ref[...], axis=0)
        o_ref[...] = jnp.maximum(g + af * h_ref[...] + b_ref[...], 0.0)

    return pl.pallas_call(
        body,
        grid=(m // bm,),
        in_specs=[
            pl.BlockSpec((npart, bm, crow), lambda i: (0, i, 0)),
            pl.BlockSpec((bm, crow), lambda i: (i, 0)),
            pl.BlockSpec((bm, 16), lambda i: (i, 0)),
            pl.BlockSpec((16, crow), lambda i: (0, 0)),
            pl.BlockSpec((1, crow), lambda i: (0, 0)),
        ],
        out_specs=pl.BlockSpec((bm, crow), lambda i: (i, 0)),
        out_shape=jax.ShapeDtypeStruct((m, crow), F32),
    )(aggP, h, asel16, SEL, bvec)


def _edge_scalars(H):
    SN = H * NPAD
    EPT = E // (NC * NS)
    NCHUNK = EPT // CH
    mesh = plsc.VectorSubcoreMesh(core_axis_name="c", subcore_axis_name="s")

    @functools.partial(
        pl.kernel,
        out_type=[
            jax.ShapeDtypeStruct((H * E,), F32),
            jax.ShapeDtypeStruct((NC * NS * SN,), F32),
        ],
        mesh=mesh,
        compiler_params=pltpu.CompilerParams(needs_layout_passes=False),
        scratch_types=[
            *([pltpu.VMEM((NPAD,), F32)] * (2 * H)),
            pltpu.VMEM((SN,), F32),
            pltpu.VMEM((CH,), I32),
            pltpu.VMEM((CH,), I32),
            *([pltpu.VMEM((CH,), F32)] * H),
        ],
    )
    def k(srcE, dstE, alphT, pE, sOUT, *rest):
        al_v = rest[:2 * H]
        sp_f, src_b, dst_b = rest[2 * H:2 * H + 3]
        p_b = rest[2 * H + 3:]
        cid = lax.axis_index("c")
        sid = lax.axis_index("s")
        wid = sid * NC + cid
        for h in range(2 * H):
            pltpu.sync_copy(alphT.at[pl.ds(h * NPAD, NPAD)], al_v[h])
        zero = jnp.zeros((L,), F32)

        def z1(i, _):
            sp_f[pl.ds(i * L, L)] = zero
            return 0
        lax.fori_loop(0, SN // L, z1, 0)

        base = wid * EPT

        def chunk(c, _):
            off = base + c * CH
            pltpu.sync_copy(srcE.at[pl.ds(off, CH)], src_b)
            pltpu.sync_copy(dstE.at[pl.ds(off, CH)], dst_b)

            def grp(i, _):
                o = pl.ds(i * L, L)
                s = src_b[o]
                d = dst_b[o]
                for h in range(H):
                    a = plsc.load_gather(al_v[h], (s,))
                    b = plsc.load_gather(al_v[H + h], (d,))
                    e = a + b
                    e = jnp.where(e >= 0.0, e, e * 0.2)
                    p = jnp.exp(e)
                    p_b[h][o] = p
                    plsc.addupdate_scatter(sp_f, (d + h * NPAD,), p)
                return 0
            lax.fori_loop(0, CH // L, grp, 0)
            for h in range(H):
                pltpu.sync_copy(p_b[h], pE.at[pl.ds(h * E + off, CH)])
            return 0
        lax.fori_loop(0, NCHUNK, chunk, 0)

        pltpu.sync_copy(sp_f, sOUT.at[pl.ds(wid * SN, SN)])

    return k


def _alpha_edges(H):
    EPT = E // (NC * NS)
    NCHUNK = EPT // CH
    mesh = plsc.VectorSubcoreMesh(core_axis_name="c", subcore_axis_name="s")

    @functools.partial(
        pl.kernel,
        out_type=jax.ShapeDtypeStruct((H * E,), F32),
        mesh=mesh,
        compiler_params=pltpu.CompilerParams(needs_layout_passes=False),
        scratch_types=[
            *([pltpu.VMEM((NPAD,), F32)] * H),
            pltpu.VMEM((CH,), I32),
            *([pltpu.VMEM((CH,), F32)] * H),
        ],
    )
    def k(dstE, pE, rinvT, aE, *rest):
        rv_v = rest[:H]
        dst_b = rest[H]
        p_b = rest[H + 1:]
        cid = lax.axis_index("c")
        sid = lax.axis_index("s")
        for h in range(H):
            pltpu.sync_copy(rinvT.at[pl.ds(h * NPAD, NPAD)], rv_v[h])
        base = (sid * NC + cid) * EPT

        def chunk(c, _):
            off = base + c * CH
            pltpu.sync_copy(dstE.at[pl.ds(off, CH)], dst_b)
            for h in range(H):
                pltpu.sync_copy(pE.at[pl.ds(h * E + off, CH)], p_b[h])

            def grp(i, _):
                o = pl.ds(i * L, L)
                d = dst_b[o]
                for h in range(H):
                    p_b[h][o] = p_b[h][o] * plsc.load_gather(rv_v[h], (d,))
                return 0
            lax.fori_loop(0, CH // L, grp, 0)
            for h in range(H):
                pltpu.sync_copy(p_b[h], aE.at[pl.ds(h * E + off, CH)])
            return 0
        lax.fori_loop(0, NCHUNK, chunk, 0)

    return k


def _edge_aggregate(H, C, SLAB, SH, GB=32, CHB=2000):
    CROW = H * C
    NV = CROW // L
    RNG = NPAD // SLAB
    EPT = E // (NC * NS)
    NCHUNK = EPT // CHB
    PKS = 20
    PKM = (1 << PKS) - 1
    mesh = plsc.VectorSubcoreMesh(core_axis_name="c", subcore_axis_name="s")

    @functools.partial(
        pl.kernel,
        out_type=jax.ShapeDtypeStruct((NC * NS * NPAD, CROW), F32),
        mesh=mesh,
        compiler_params=pltpu.CompilerParams(needs_layout_passes=False),
        scratch_types=[
            pltpu.VMEM((CHB,), I32),
            pltpu.VMEM((CHB,), I32),
            *([pltpu.VMEM((CHB,), F32)] * H),
            pltpu.VMEM((EPT + GB,), I32),
            *([pltpu.VMEM((EPT,), F32)] * H),
            pltpu.VMEM((RNG,), I32),
            pltpu.VMEM((RNG + L,), I32),
            pltpu.VMEM((RNG + L,), I32),
            pltpu.VMEM((GB,), I32),
            pltpu.VMEM((SLAB, CROW), F32),
            pltpu.VMEM((GB, CROW), F32),
            pltpu.SemaphoreType.DMA,
        ],
    )
    def k(srcE, dstE, aE, feat, aggP, *rest):
        src_b, dst_b = rest[0:2]
        a_b = rest[2:2 + H]
        srcS = rest[2 + H]
        aS = rest[3 + H:3 + 2 * H]
        cnt, st, wk, gi_v, slab, gst, sem = rest[3 + 2 * H:]
        cid = lax.axis_index("c")
        sid = lax.axis_index("s")
        wid = sid * NC + cid
        base = wid * EPT
        zero = jnp.zeros((L,), F32)
        zeroi = jnp.zeros((L,), I32)
        onei = jnp.ones((L,), I32)

        def zc(i, _):
            cnt[pl.ds(i * L, L)] = zeroi
            return 0
        lax.fori_loop(0, RNG // L, zc, 0)
        for q in range(GB // L):
            srcS[pl.ds(EPT + q * L, L)] = zeroi

        def chunk1(c, _):
            off = base + c * CHB
            pltpu.sync_copy(dstE.at[pl.ds(off, CHB)], dst_b)

            def grp(i, _):
                d = dst_b[pl.ds(i * L, L)]
                plsc.addupdate_scatter(cnt, (lax.shift_right_logical(d, SH),),
                                       onei)
                return 0
            lax.fori_loop(0, CHB // L, grp, 0)
            return 0
        lax.fori_loop(0, NCHUNK, chunk1, 0)

        def cs(g, acc):
            o = pl.ds(g * L, L)
            v = cnt[o]
            inc = plsc.cumsum(v)
            exc = inc - v + acc
            st[o] = exc
            wk[o] = exc
            return acc + jnp.max(inc)
        tot = lax.fori_loop(0, RNG // L, cs, jnp.int32(0))
        st[pl.ds(RNG, L)] = zeroi + tot

        def chunk2(c, _):
            off = base + c * CHB
            pltpu.sync_copy(srcE.at[pl.ds(off, CHB)], src_b)
            pltpu.sync_copy(dstE.at[pl.ds(off, CHB)], dst_b)
            for h in range(H):
                pltpu.sync_copy(aE.at[pl.ds(h * E + off, CHB)], a_b[h])

            def place(i, _):
                o = pl.ds(i * L, L)
                d16 = dst_b[o]
                s16 = src_b[o]
                a16 = [a_b[h][o] for h in range(H)]
                rg16 = lax.shift_right_logical(d16, SH)
                pk16 = jnp.bitwise_or(
                    s16, lax.shift_left(lax.bitwise_and(d16, SLAB - 1), PKS))
                for lane in range(L):
                    rg = rg16[lane]
                    po = wk[pl.ds(rg, L)][0]
                    pov = jnp.full((L,), po, I32)
                    plsc.store_scatter(wk, (jnp.full((L,), rg, I32),),
                                       pov + 1)
                    plsc.store_scatter(srcS, (pov,),
                                       jnp.full((L,), pk16[lane], I32))
                    for h in range(H):
                        plsc.store_scatter(aS[h], (pov,),
                                           jnp.full((L,), a16[h][lane], F32))
                return 0
            lax.fori_loop(0, CHB // L, place, 0)
            return 0
        lax.fori_loop(0, NCHUNK, chunk2, 0)

        def rng_body(r, _):
            def zs(q, _):
                for v in range(NV):
                    slab[q, pl.ds(v * L, L)] = zero
                return 0
            lax.fori_loop(0, SLAB, zs, 0)
            j0 = st[pl.ds(r, L)][0]
            j1 = st[pl.ds(r + 1, L)][0]
            nb = (j1 - j0 + GB - 1) // GB

            def batch(b, _):
                g0 = j0 + b * GB
                for q in range(GB // L):
                    gi_v[pl.ds(q * L, L)] = jnp.bitwise_and(
                        srcS[pl.ds(g0 + q * L, L)], PKM)
                pltpu.async_copy(feat.at[gi_v], gst, sem).wait()
                njj = jnp.minimum(jnp.int32(GB), j1 - g0)

                def rec(jj, _):
                    j = g0 + jj
                    dl = lax.shift_right_logical(srcS[pl.ds(j, L)][0], PKS)
                    for h in range(H):
                        asp = plsc.load_gather(aS[h],
                                               (jnp.full((L,), j, I32),))
                        for v in range(C // L):
                            oo = pl.ds((h * (C // L) + v) * L, L)
                            slab[dl, oo] = slab[dl, oo] + gst[jj, oo] * asp
                    return 0
                lax.fori_loop(0, njj, rec, 0)
                return 0
            lax.fori_loop(0, nb, batch, 0)
            pltpu.sync_copy(slab,
                            aggP.at[pl.ds(wid * NPAD + r * SLAB, SLAB)])
            return 0
        lax.fori_loop(0, RNG, rng_body, 0)

    return k


_PASS_A1 = _edge_scalars(HEADS)
_PASS_A2 = _edge_scalars(1)
_ALPHA1 = _alpha_edges(HEADS)
_ALPHA2 = _alpha_edges(1)
_PASS_B1 = _edge_aggregate(HEADS, HID, 64, 6)
_PASS_B2 = _edge_aggregate(1, OUT, 128, 7)


def _sel_matrix(H, C):
    m = jnp.zeros((16, H * C), F32)
    for h in range(H):
        m = m.at[h, h * C:(h + 1) * C].set(1.0)
    return m


def _layer(srcE, dstE, xin, W, Amat, bvec, H, C, pass_a, alpha_p, pass_b):
    crow = H * C
    h, al = _mm_alpha(xin, W, Amat)
    alphT = al[:, :2 * H].T
    p, s = pass_a(srcE, dstE, alphT.reshape(-1))
    s32 = s.reshape(NC * NS, H, NPAD)
    rinvT, aselT = _rinv_self(s32, alphT[:H], alphT[H:])
    aE = alpha_p(dstE, p, rinvT.reshape(-1))
    aggP = pass_b(srcE, dstE, aE, h)
    asel16 = jnp.zeros((16, NPAD), F32).at[:H].set(aselT).T
    out = _combine(aggP.reshape(NC * NS, NPAD, crow), h, asel16,
                   bvec.reshape(1, crow), _sel_matrix(H, C))
    return out


def kernel(x, edge_index, W1, a_src1, a_dst1, b1, W2, a_src2, a_dst2, b2):
    ei = edge_index.astype(I32)
    srcE, dstE = ei[0], ei[1]
    xp = jnp.zeros((NPAD, IN), F32).at[:N].set(x)

    A1 = jnp.zeros((HEADS * HID, 16), F32)
    for h in range(HEADS):
        A1 = A1.at[h * HID:(h + 1) * HID, h].set(a_src1[h])
        A1 = A1.at[h * HID:(h + 1) * HID, HEADS + h].set(a_dst1[h])
    A2 = jnp.zeros((OUT, 16), F32).at[:, 0].set(a_src2[0]).at[:, 1].set(a_dst2[0])

    out1 = _layer(srcE, dstE, xp, W1, A1, b1, HEADS, HID,
                  _PASS_A1, _ALPHA1, _PASS_B1)
    out2 = _layer(srcE, dstE, out1, W2, A2, b2, 1, OUT,
                  _PASS_A2, _ALPHA2, _PASS_B2)
    return out2[:N]


# speedup vs baseline: 1.1192x; 1.1192x over previous
import functools

import jax
import jax.numpy as jnp
from jax import lax
from jax.experimental import pallas as pl
from jax.experimental.pallas import tpu as pltpu
from jax.experimental.pallas import tpu_sc as plsc

N = 10000
E = 320000
IN = 128
HID = 256
OUT = 384
HEADS = 3
NPAD = 12288
NC, NS, L = 2, 16, 16
CH = 2000
FB = 64
F32 = jnp.float32
I32 = jnp.int32


def _mm_alpha(x, W, Amat, bm=1024):
    m, k = x.shape
    c = W.shape[1]

    def body(x_ref, w_ref, a_ref, y_ref, al_ref):
        h = jnp.dot(x_ref[...], w_ref[...], preferred_element_type=F32)
        y_ref[...] = h
        al_ref[...] = jnp.dot(h, a_ref[...], preferred_element_type=F32)

    return pl.pallas_call(
        body,
        grid=(m // bm,),
        in_specs=[
            pl.BlockSpec((bm, k), lambda i: (i, 0)),
            pl.BlockSpec((k, c), lambda i: (0, 0)),
            pl.BlockSpec((c, 16), lambda i: (0, 0)),
        ],
        out_specs=[
            pl.BlockSpec((bm, c), lambda i: (i, 0)),
            pl.BlockSpec((bm, 16), lambda i: (i, 0)),
        ],
        out_shape=[
            jax.ShapeDtypeStruct((m, c), F32),
            jax.ShapeDtypeStruct((m, 16), F32),
        ],
    )(x, W, Amat)


def _rinv_self(s32, al_a, al_b):
    H = al_a.shape[0]

    def body(s_ref, aa_ref, ab_ref, rv_ref, as_ref):
        es = aa_ref[...] + ab_ref[...]
        ps = jnp.exp(jnp.where(es >= 0.0, es, es * 0.2))
        st = jnp.sum(s_ref[...], axis=0) + ps
        rv = 1.0 / (st + 1e-16)
        rv_ref[...] = rv
        as_ref[...] = ps * rv

    return pl.pallas_call(
        body,
        out_shape=[
            jax.ShapeDtypeStruct((H, NPAD), F32),
            jax.ShapeDtypeStruct((H, NPAD), F32),
        ],
    )(s32, al_a, al_b)


def _combine(aggP, h, asel16, bvec, SEL, bm=128):
    m, crow = h.shape
    npart = aggP.shape[0]

    def body(g_ref, h_ref, a_ref, s_ref, b_ref, o_ref):
        af = jnp.dot(a_ref[...], s_ref[...], preferred_element_type=F32)
        g = jnp.sum(g_ref[...], axis=0)
        o_ref[...] = jnp.maximum(g + af * h_ref[...] + b_ref[...], 0.0)

    return pl.pallas_call(
        body,
        grid=(m // bm,),
        in_specs=[
            pl.BlockSpec((npart, bm, crow), lambda i: (0, i, 0)),
            pl.BlockSpec((bm, crow), lambda i: (i, 0)),
            pl.BlockSpec((bm, 16), lambda i: (i, 0)),
            pl.BlockSpec((16, crow), lambda i: (0, 0)),
            pl.BlockSpec((1, crow), lambda i: (0, 0)),
        ],
        out_specs=pl.BlockSpec((bm, crow), lambda i: (i, 0)),
        out_shape=jax.ShapeDtypeStruct((m, crow), F32),
    )(aggP, h, asel16, SEL, bvec)


def _edge_scalars(H):
    SN = H * NPAD
    EPT = E // (NC * NS)
    NCHUNK = EPT // CH
    mesh = plsc.VectorSubcoreMesh(core_axis_name="c", subcore_axis_name="s")

    @functools.partial(
        pl.kernel,
        out_type=[
            jax.ShapeDtypeStruct((H * E,), F32),
            jax.ShapeDtypeStruct((NC * NS * SN,), F32),
        ],
        mesh=mesh,
        compiler_params=pltpu.CompilerParams(needs_layout_passes=False),
        scratch_types=[
            *([pltpu.VMEM((NPAD,), F32)] * (2 * H)),
            pltpu.VMEM((SN,), F32),
            pltpu.VMEM((CH,), I32),
            pltpu.VMEM((CH,), I32),
            *([pltpu.VMEM((CH,), F32)] * H),
        ],
    )
    def k(srcE, dstE, alphT, pE, sOUT, *rest):
        al_v = rest[:2 * H]
        sp_f, src_b, dst_b = rest[2 * H:2 * H + 3]
        p_b = rest[2 * H + 3:]
        cid = lax.axis_index("c")
        sid = lax.axis_index("s")
        wid = sid * NC + cid
        for h in range(2 * H):
            pltpu.sync_copy(alphT.at[pl.ds(h * NPAD, NPAD)], al_v[h])
        zero = jnp.zeros((L,), F32)

        def z1(i, _):
            sp_f[pl.ds(i * L, L)] = zero
            return 0
        lax.fori_loop(0, SN // L, z1, 0)

        base = wid * EPT

        def chunk(c, _):
            off = base + c * CH
            pltpu.sync_copy(srcE.at[pl.ds(off, CH)], src_b)
            pltpu.sync_copy(dstE.at[pl.ds(off, CH)], dst_b)

            def grp(i, _):
                o = pl.ds(i * L, L)
                s = src_b[o]
                d = dst_b[o]
                for h in range(H):
                    a = plsc.load_gather(al_v[h], (s,))
                    b = plsc.load_gather(al_v[H + h], (d,))
                    e = a + b
                    e = jnp.where(e >= 0.0, e, e * 0.2)
                    p = jnp.exp(e)
                    p_b[h][o] = p
                    plsc.addupdate_scatter(sp_f, (d + h * NPAD,), p)
                return 0
            lax.fori_loop(0, CH // L, grp, 0)
            for h in range(H):
                pltpu.sync_copy(p_b[h], pE.at[pl.ds(h * E + off, CH)])
            return 0
        lax.fori_loop(0, NCHUNK, chunk, 0)

        pltpu.sync_copy(sp_f, sOUT.at[pl.ds(wid * SN, SN)])

    return k


def _alpha_edges(H):
    EPT = E // (NC * NS)
    NCHUNK = EPT // CH
    mesh = plsc.VectorSubcoreMesh(core_axis_name="c", subcore_axis_name="s")

    @functools.partial(
        pl.kernel,
        out_type=jax.ShapeDtypeStruct((H * E,), F32),
        mesh=mesh,
        compiler_params=pltpu.CompilerParams(needs_layout_passes=False),
        scratch_types=[
            *([pltpu.VMEM((NPAD,), F32)] * H),
            pltpu.VMEM((CH,), I32),
            *([pltpu.VMEM((CH,), F32)] * H),
        ],
    )
    def k(dstE, pE, rinvT, aE, *rest):
        rv_v = rest[:H]
        dst_b = rest[H]
        p_b = rest[H + 1:]
        cid = lax.axis_index("c")
        sid = lax.axis_index("s")
        for h in range(H):
            pltpu.sync_copy(rinvT.at[pl.ds(h * NPAD, NPAD)], rv_v[h])
        base = (sid * NC + cid) * EPT

        def chunk(c, _):
            off = base + c * CH
            pltpu.sync_copy(dstE.at[pl.ds(off, CH)], dst_b)
            for h in range(H):
                pltpu.sync_copy(pE.at[pl.ds(h * E + off, CH)], p_b[h])

            def grp(i, _):
                o = pl.ds(i * L, L)
                d = dst_b[o]
                for h in range(H):
                    p_b[h][o] = p_b[h][o] * plsc.load_gather(rv_v[h], (d,))
                return 0
            lax.fori_loop(0, CH // L, grp, 0)
            for h in range(H):
                pltpu.sync_copy(p_b[h], aE.at[pl.ds(h * E + off, CH)])
            return 0
        lax.fori_loop(0, NCHUNK, chunk, 0)

    return k


def _edge_aggregate(H, C, SLAB, SH, GB=16, CHB=2000):
    CROW = H * C
    NV = CROW // L
    RNG = NPAD // SLAB
    EPT = E // (NC * NS)
    NCHUNK = EPT // CHB
    PKS = 20
    PKM = (1 << PKS) - 1
    mesh = plsc.VectorSubcoreMesh(core_axis_name="c", subcore_axis_name="s")

    @functools.partial(
        pl.kernel,
        out_type=jax.ShapeDtypeStruct((NC * NS * NPAD, CROW), F32),
        mesh=mesh,
        compiler_params=pltpu.CompilerParams(needs_layout_passes=False),
        scratch_types=[
            pltpu.VMEM((CHB,), I32),
            pltpu.VMEM((CHB,), I32),
            *([pltpu.VMEM((CHB,), F32)] * H),
            pltpu.VMEM((EPT + GB,), I32),
            *([pltpu.VMEM((EPT,), F32)] * H),
            pltpu.VMEM((RNG,), I32),
            pltpu.VMEM((RNG + L,), I32),
            pltpu.VMEM((RNG + L,), I32),
            pltpu.VMEM((GB,), I32),
            pltpu.VMEM((GB,), I32),
            pltpu.VMEM((SLAB, CROW), F32),
            pltpu.VMEM((GB, CROW), F32),
            pltpu.VMEM((GB, CROW), F32),
            pltpu.SemaphoreType.DMA,
            pltpu.SemaphoreType.DMA,
        ],
    )
    def k(srcE, dstE, aE, feat, aggP, *rest):
        src_b, dst_b = rest[0:2]
        a_b = rest[2:2 + H]
        srcS = rest[2 + H]
        aS = rest[3 + H:3 + 2 * H]
        cnt, st, wk, gi_a, gi_b, slab, gstA, gstB, semA, semB = \
            rest[3 + 2 * H:]
        cid = lax.axis_index("c")
        sid = lax.axis_index("s")
        wid = sid * NC + cid
        base = wid * EPT
        zero = jnp.zeros((L,), F32)
        zeroi = jnp.zeros((L,), I32)
        onei = jnp.ones((L,), I32)

        def zc(i, _):
            cnt[pl.ds(i * L, L)] = zeroi
            return 0
        lax.fori_loop(0, RNG // L, zc, 0)
        for q in range(GB // L):
            srcS[pl.ds(EPT + q * L, L)] = zeroi

        def chunk1(c, _):
            off = base + c * CHB
            pltpu.sync_copy(dstE.at[pl.ds(off, CHB)], dst_b)

            def grp(i, _):
                d = dst_b[pl.ds(i * L, L)]
                plsc.addupdate_scatter(cnt, (lax.shift_right_logical(d, SH),),
                                       onei)
                return 0
            lax.fori_loop(0, CHB // L, grp, 0)
            return 0
        lax.fori_loop(0, NCHUNK, chunk1, 0)

        def cs(g, acc):
            o = pl.ds(g * L, L)
            v = cnt[o]
            inc = plsc.cumsum(v)
            exc = inc - v + acc
            st[o] = exc
            wk[o] = exc
            return acc + jnp.max(inc)
        tot = lax.fori_loop(0, RNG // L, cs, jnp.int32(0))
        st[pl.ds(RNG, L)] = zeroi + tot

        def chunk2(c, _):
            off = base + c * CHB
            pltpu.sync_copy(srcE.at[pl.ds(off, CHB)], src_b)
            pltpu.sync_copy(dstE.at[pl.ds(off, CHB)], dst_b)
            for h in range(H):
                pltpu.sync_copy(aE.at[pl.ds(h * E + off, CHB)], a_b[h])

            def place(i, _):
                o = pl.ds(i * L, L)
                d16 = dst_b[o]
                s16 = src_b[o]
                a16 = [a_b[h][o] for h in range(H)]
                rg16 = lax.shift_right_logical(d16, SH)
                pk16 = jnp.bitwise_or(
                    s16, lax.shift_left(lax.bitwise_and(d16, SLAB - 1), PKS))
                for lane in range(L):
                    rg = rg16[lane]
                    po = wk[pl.ds(rg, L)][0]
                    pov = jnp.full((L,), po, I32)
                    plsc.store_scatter(wk, (jnp.full((L,), rg, I32),),
                                       pov + 1)
                    plsc.store_scatter(srcS, (pov,),
                                       jnp.full((L,), pk16[lane], I32))
                    for h in range(H):
                        plsc.store_scatter(aS[h], (pov,),
                                           jnp.full((L,), a16[h][lane], F32))
                return 0
            lax.fori_loop(0, CHB // L, place, 0)
            return 0
        lax.fori_loop(0, NCHUNK, chunk2, 0)

        def rng_body(r, _):
            def zs(q, _):
                for v in range(NV):
                    slab[q, pl.ds(v * L, L)] = zero
                return 0
            lax.fori_loop(0, SLAB, zs, 0)
            j0 = st[pl.ds(r, L)][0]
            j1 = st[pl.ds(r + 1, L)][0]
            nb = (j1 - j0 + GB - 1) // GB

            def fill(gi, b):
                g0 = j0 + b * GB
                for q in range(GB // L):
                    gi[pl.ds(q * L, L)] = jnp.bitwise_and(
                        srcS[pl.ds(g0 + q * L, L)], PKM)

            def proc(gstX, b):
                g0 = j0 + b * GB
                njj = jnp.minimum(jnp.int32(GB), j1 - g0)

                def rec(jj, _):
                    j = g0 + jj
                    dl = lax.shift_right_logical(srcS[pl.ds(j, L)][0], PKS)
                    for h in range(H):
                        asp = plsc.load_gather(aS[h],
                                               (jnp.full((L,), j, I32),))
                        for v in range(C // L):
                            oo = pl.ds((h * (C // L) + v) * L, L)
                            slab[dl, oo] = slab[dl, oo] + gstX[jj, oo] * asp
                    return 0
                lax.fori_loop(0, njj, rec, 0)

            @pl.when(nb > 0)
            def _():
                fill(gi_a, jnp.int32(0))
                pltpu.async_copy(feat.at[gi_a], gstA, semA)

            def batch(b, _):
                even = lax.rem(b, 2) == 0

                @pl.when(even)
                def _():
                    pltpu.make_async_copy(feat.at[gi_a], gstA, semA).wait()

                    @pl.when(b + 1 < nb)
                    def _():
                        fill(gi_b, b + 1)
                        pltpu.async_copy(feat.at[gi_b], gstB, semB)
                    proc(gstA, b)

                @pl.when(jnp.logical_not(even))
                def _():
                    pltpu.make_async_copy(feat.at[gi_b], gstB, semB).wait()

                    @pl.when(b + 1 < nb)
                    def _():
                        fill(gi_a, b + 1)
                        pltpu.async_copy(feat.at[gi_a], gstA, semA)
                    proc(gstB, b)
                return 0
            lax.fori_loop(0, nb, batch, 0)
            pltpu.sync_copy(slab,
                            aggP.at[pl.ds(wid * NPAD + r * SLAB, SLAB)])
            return 0
        lax.fori_loop(0, RNG, rng_body, 0)

    return k


_PASS_A1 = _edge_scalars(HEADS)
_PASS_A2 = _edge_scalars(1)
_ALPHA1 = _alpha_edges(HEADS)
_ALPHA2 = _alpha_edges(1)
_PASS_B1 = _edge_aggregate(HEADS, HID, 64, 6)
_PASS_B2 = _edge_aggregate(1, OUT, 128, 7)


def _sel_matrix(H, C):
    m = jnp.zeros((16, H * C), F32)
    for h in range(H):
        m = m.at[h, h * C:(h + 1) * C].set(1.0)
    return m


def _layer(srcE, dstE, xin, W, Amat, bvec, H, C, pass_a, alpha_p, pass_b):
    crow = H * C
    h, al = _mm_alpha(xin, W, Amat)
    alphT = al[:, :2 * H].T
    p, s = pass_a(srcE, dstE, alphT.reshape(-1))
    s32 = s.reshape(NC * NS, H, NPAD)
    rinvT, aselT = _rinv_self(s32, alphT[:H], alphT[H:])
    aE = alpha_p(dstE, p, rinvT.reshape(-1))
    aggP = pass_b(srcE, dstE, aE, h)
    asel16 = jnp.zeros((16, NPAD), F32).at[:H].set(aselT).T
    out = _combine(aggP.reshape(NC * NS, NPAD, crow), h, asel16,
                   bvec.reshape(1, crow), _sel_matrix(H, C))
    return out


def kernel(x, edge_index, W1, a_src1, a_dst1, b1, W2, a_src2, a_dst2, b2):
    ei = edge_index.astype(I32)
    srcE, dstE = ei[0], ei[1]
    xp = jnp.zeros((NPAD, IN), F32).at[:N].set(x)

    A1 = jnp.zeros((HEADS * HID, 16), F32)
    for h in range(HEADS):
        A1 = A1.at[h * HID:(h + 1) * HID, h].set(a_src1[h])
        A1 = A1.at[h * HID:(h + 1) * HID, HEADS + h].set(a_dst1[h])
    A2 = jnp.zeros((OUT, 16), F32).at[:, 0].set(a_src2[0]).at[:, 1].set(a_dst2[0])

    out1 = _layer(srcE, dstE, xp, W1, A1, b1, HEADS, HID,
                  _PASS_A1, _ALPHA1, _PASS_B1)
    out2 = _layer(srcE, dstE, out1, W2, A2, b2, 1, OUT,
                  _PASS_A2, _ALPHA2, _PASS_B2)
    return out2[:N]
